# baseline (speedup 1.0000x reference)
import functools

import jax
import jax.numpy as jnp
from jax import lax
from jax.experimental import pallas as pl
from jax.experimental.pallas import tpu as pltpu

_BLK = 1024


def _tc_embed(idx3, table_hl):
    nb = idx3.shape[0]
    vocab, two_d = table_hl.shape
    embed_dim = two_d // 2
    rows = _BLK * embed_dim // 128
    rpc = 128 // embed_dim

    def body(idx_ref, tab_ref, out_ref):
        idx_t = lax.transpose(idx_ref[0], (1, 0))
        vio = lax.broadcasted_iota(jnp.int32, (rows, vocab), 1)
        for c in range(rpc):
            col = jnp.concatenate(
                [idx_t[:, 2 * c:2 * c + 1], idx_t[:, 2 * c + 1:2 * c + 2]],
                axis=0)
            oh = (col == vio).astype(jnp.bfloat16)
            r = lax.dot_general(oh, tab_ref[...], (((1,), (0,)), ((), ())),
                                preferred_element_type=jnp.float32)
            out_ref[:, pl.ds(embed_dim * c, embed_dim)] = (
                r[:, :embed_dim] + r[:, embed_dim:])

    return pl.pallas_call(
        body,
        grid=(nb,),
        in_specs=[
            pl.BlockSpec((1, 8, 128), lambda i: (i, 0, 0)),
            pl.BlockSpec((vocab, two_d), lambda i: (0, 0)),
        ],
        out_specs=pl.BlockSpec((rows, 128), lambda i: (i, 0)),
        out_shape=jax.ShapeDtypeStruct((nb * rows, 128), jnp.float32),
    )(idx3, table_hl)


def kernel(indices, table):
    batch, hist = indices.shape
    vocab, embed_dim = table.shape
    n = batch * hist
    nb = n // _BLK
    th = table.astype(jnp.bfloat16)
    tl = (table - th.astype(jnp.float32)).astype(jnp.bfloat16)
    table_hl = jnp.concatenate([th, tl], axis=1)
    idx3 = (indices.reshape(nb, 2, 128, 4).transpose(0, 3, 1, 2)
            .reshape(nb, 8, 128))
    out = _tc_embed(idx3, table_hl)
    return out.reshape(batch, hist, embed_dim)


# speedup vs baseline: 2.2696x; 2.2696x over previous
import functools

import jax
import jax.numpy as jnp
from jax import lax
from jax.experimental import pallas as pl
from jax.experimental.pallas import tpu as pltpu
from jax.experimental.pallas import tpu_sc as plsc

_NC = 2
_NS = 16
_NW = _NC * _NS


def _embed_gather(idx_grouped, table, *, niter, chunk, embed_dim):
    n_rows = _NW * niter * chunk
    mesh = plsc.VectorSubcoreMesh(core_axis_name="c", subcore_axis_name="s")

    @functools.partial(
        pl.kernel,
        mesh=mesh,
        out_type=jax.ShapeDtypeStruct((n_rows, embed_dim), jnp.float32),
        scratch_types=[
            pltpu.VMEM((niter, chunk), jnp.int32),
            pltpu.VMEM_SHARED(table.shape, jnp.float32),
            pltpu.VMEM((chunk, embed_dim), jnp.float32),
            pltpu.VMEM((chunk, embed_dim), jnp.float32),
            pltpu.SemaphoreType.DMA,
            pltpu.SemaphoreType.DMA,
        ],
        compiler_params=pltpu.CompilerParams(use_tc_tiling_on_sc=False),
    )
    def k(idx_hbm, table_hbm, out_hbm, idx_v, table_v, rows0, rows1, sem0,
          sem1):
        sid = lax.axis_index("s")
        wid = sid * _NC + lax.axis_index("c")

        @pl.when(sid == 0)
        def _():
            pltpu.sync_copy(table_hbm, table_v)

        pltpu.sync_copy(idx_hbm.at[wid], idx_v)
        plsc.subcore_barrier()

        def gather(i, buf, sem):
            return pltpu.async_copy(table_v.at[idx_v.at[i]], buf, sem)

        def wait_gather(i, buf, sem):
            pltpu.make_async_copy(table_v.at[idx_v.at[i]], buf, sem).wait()

        def scatter(i, buf):
            base = (wid * niter + i) * chunk
            pltpu.sync_copy(buf, out_hbm.at[pl.ds(base, chunk)])

        gather(0, rows0, sem0)

        def step2(j, carry):
            i0 = 2 * j
            gather(i0 + 1, rows1, sem1)
            wait_gather(i0, rows0, sem0)
            scatter(i0, rows0)

            @pl.when(j + 1 < niter // 2)
            def _():
                gather(i0 + 2, rows0, sem0)

            wait_gather(i0 + 1, rows1, sem1)
            scatter(i0 + 1, rows1)
            return carry

        lax.fori_loop(0, niter // 2, step2, 0)

    return k(idx_grouped, table)


def kernel(indices, table):
    batch, hist = indices.shape
    vocab, embed_dim = table.shape
    n = batch * hist
    chunk = 1600
    niter = n // (_NW * chunk)
    idx_grouped = indices.reshape(_NW, niter, chunk)
    out = _embed_gather(idx_grouped, table, niter=niter, chunk=chunk,
                        embed_dim=embed_dim)
    return out.reshape(batch, hist, embed_dim)
